# baseline (speedup 1.0000x reference)
import jax
import jax.numpy as jnp
import numpy as np
from jax import lax
from jax.experimental import pallas as pl
from jax.experimental.pallas import tpu as pltpu
from jax.experimental.pallas import tpu_sc as plsc

_D = 512
_B = 32
_S = 2048
_NC = 2
_NS = 16
_NW = _NC * _NS
_PPW = _S // _NW
_LANES = 16
_CHUNK = 16
_NSTEP = _B * _PPW // _CHUNK
_NBUF = 8
_DIST = 4


def _positional_table():
    pos = np.arange(0, _S, dtype=np.float32)[:, None]
    _2i = np.arange(0, _D, 2, dtype=np.float32)
    angle = pos / np.power(np.float32(1000.0), _2i / np.float32(_D))
    pe = np.zeros((_S, _D), dtype=np.float32)
    pe[:, 0::2] = np.sin(angle)
    pe[:, 1::2] = np.cos(angle)
    return pe


_PE = _positional_table()


def _emb_body(x_hbm, pe_hbm, table_hbm, out_hbm, *refs):
    idx_v, pe_v = refs[0], refs[1]
    bufs = refs[2:2 + _NBUF]
    gsem = refs[2 + _NBUF:2 + 2 * _NBUF]
    ssem = refs[2 + 2 * _NBUF:2 + 3 * _NBUF]
    isem, psem = refs[2 + 3 * _NBUF], refs[3 + 3 * _NBUF]
    w = lax.axis_index("s") * _NC + lax.axis_index("c")
    p0 = w * _PPW

    pltpu.async_copy(pe_hbm.at[pl.ds(p0, _PPW)], pe_v, psem)

    @pl.loop(0, _B)
    def _idx(b):
        pltpu.async_copy(x_hbm.at[pl.ds(b * _S + p0, _PPW)], idx_v.at[b],
                         isem)

    @pl.loop(0, _B)
    def _idx_drain(b):
        pltpu.make_async_copy(x_hbm.at[pl.ds(0, _PPW)], idx_v.at[b],
                              isem).wait()

    _CPT = _PPW // _CHUNK

    def start_gather(k, slot):
        b = k // _CPT
        h = k % _CPT
        pltpu.async_copy(
            table_hbm.at[idx_v.at[b, pl.ds(h * _CHUNK, _CHUNK)]],
            bufs[slot], gsem[slot])

    def start_scatter(k, slot):
        b = k // _CPT
        h = k % _CPT
        base = b * _S + p0 + h * _CHUNK
        pltpu.async_copy(bufs[slot], out_hbm.at[pl.ds(base, _CHUNK)],
                         ssem[slot])

    def wait_bytes(sem, slot):
        pltpu.make_async_copy(out_hbm.at[pl.ds(0, _CHUNK)], bufs[slot],
                              sem[slot]).wait()

    for s in range(_DIST):
        start_gather(s, s)

    pltpu.make_async_copy(pe_hbm.at[pl.ds(0, _PPW)], pe_v, psem).wait()

    @pl.loop(0, _NSTEP, step=_NBUF)
    def _step(g):
        for s in range(_NBUF):
            k = g + s
            wait_bytes(gsem, s)
            h32 = (k % _CPT) * _CHUNK
            s2 = (s + _DIST) % _NBUF

            @pl.when(k + _DIST < _NSTEP)
            def _prefetch():
                @pl.when(k >= _NBUF - _DIST)
                def _reuse():
                    wait_bytes(ssem, s2)
                start_gather(k + _DIST, s2)

            @plsc.parallel_loop(0, _CHUNK, 1, unroll=4)
            def _row(r):
                pr = h32 + r
                for c in range(_D // _LANES):
                    slc = pl.ds(c * _LANES, _LANES)
                    plsc.addupdate(bufs[s].at[r, slc], pe_v[pr, slc])

            start_scatter(k, s)

    for s in range(_NBUF):
        wait_bytes(ssem, s)


@jax.jit
def kernel(x, tok_emb_table):
    mesh = plsc.VectorSubcoreMesh(core_axis_name="c", subcore_axis_name="s")
    run = pl.kernel(
        _emb_body,
        out_type=jax.ShapeDtypeStruct((_B * _S, _D), jnp.float32),
        mesh=mesh,
        scratch_types=(
            [pltpu.VMEM((_B, _PPW), jnp.int32),
             pltpu.VMEM((_PPW, _D), jnp.float32)]
            + [pltpu.VMEM((_CHUNK, _D), jnp.float32)] * _NBUF
            + [pltpu.SemaphoreType.DMA] * (2 * _NBUF + 2)
        ),
    )
    out = run(x.astype(jnp.int32).reshape(-1), jnp.asarray(_PE),
              tok_emb_table)
    return out.reshape(_B, _S, _D)


# speedup vs baseline: 1.4919x; 1.4919x over previous
import jax
import jax.numpy as jnp
import numpy as np
from jax import lax
from jax.experimental import pallas as pl
from jax.experimental.pallas import tpu as pltpu
from jax.experimental.pallas import tpu_sc as plsc

_D = 512
_B = 32
_S = 2048
_NC = 2
_NS = 16
_NW = _NC * _NS
_PPW = _S // _NW
_LANES = 16
_CHUNK = 64
_NSTEP = _B * _PPW // _CHUNK
_NBUF = 2
_DIST = 1


def _positional_table():
    pos = np.arange(0, _S, dtype=np.float32)[:, None]
    _2i = np.arange(0, _D, 2, dtype=np.float32)
    angle = pos / np.power(np.float32(1000.0), _2i / np.float32(_D))
    pe = np.zeros((_S, _D), dtype=np.float32)
    pe[:, 0::2] = np.sin(angle)
    pe[:, 1::2] = np.cos(angle)
    return pe


_PE = _positional_table()


def _emb_body(x_hbm, pe_hbm, table_hbm, out_hbm, *refs):
    idx_v, pe_v = refs[0], refs[1]
    bufs = refs[2:2 + _NBUF]
    gsem = refs[2 + _NBUF:2 + 2 * _NBUF]
    ssem = refs[2 + 2 * _NBUF:2 + 3 * _NBUF]
    isem, psem = refs[2 + 3 * _NBUF], refs[3 + 3 * _NBUF]
    w = lax.axis_index("s") * _NC + lax.axis_index("c")
    p0 = w * _PPW

    pltpu.async_copy(pe_hbm.at[pl.ds(p0, _PPW)], pe_v, psem)

    @pl.loop(0, _B)
    def _idx(b):
        pltpu.async_copy(x_hbm.at[pl.ds(b * _S + p0, _PPW)], idx_v.at[b],
                         isem)

    @pl.loop(0, _B)
    def _idx_drain(b):
        pltpu.make_async_copy(x_hbm.at[pl.ds(0, _PPW)], idx_v.at[b],
                              isem).wait()

    _CPT = _PPW // _CHUNK

    def start_gather(k, slot):
        b = k // _CPT
        h = k % _CPT
        pltpu.async_copy(
            table_hbm.at[idx_v.at[b, pl.ds(h * _CHUNK, _CHUNK)]],
            bufs[slot], gsem[slot])

    def start_scatter(k, slot):
        b = k // _CPT
        h = k % _CPT
        base = b * _S + p0 + h * _CHUNK
        pltpu.async_copy(bufs[slot], out_hbm.at[pl.ds(base, _CHUNK)],
                         ssem[slot])

    def wait_bytes(sem, slot):
        pltpu.make_async_copy(out_hbm.at[pl.ds(0, _CHUNK)], bufs[slot],
                              sem[slot]).wait()

    for s in range(_DIST):
        start_gather(s, s)

    pltpu.make_async_copy(pe_hbm.at[pl.ds(0, _PPW)], pe_v, psem).wait()

    @pl.loop(0, _NSTEP, step=_NBUF)
    def _step(g):
        for s in range(_NBUF):
            k = g + s
            wait_bytes(gsem, s)
            h32 = (k % _CPT) * _CHUNK
            s2 = (s + _DIST) % _NBUF

            @pl.when(k + _DIST < _NSTEP)
            def _prefetch():
                @pl.when(k >= _NBUF - _DIST)
                def _reuse():
                    wait_bytes(ssem, s2)
                start_gather(k + _DIST, s2)

            @plsc.parallel_loop(0, _CHUNK, 1, unroll=4)
            def _row(r):
                pr = h32 + r
                for c in range(_D // _LANES):
                    slc = pl.ds(c * _LANES, _LANES)
                    plsc.addupdate(bufs[s].at[r, slc], pe_v[pr, slc])

            start_scatter(k, s)

    for s in range(_NBUF):
        wait_bytes(ssem, s)


@jax.jit
def kernel(x, tok_emb_table):
    mesh = plsc.VectorSubcoreMesh(core_axis_name="c", subcore_axis_name="s")
    run = pl.kernel(
        _emb_body,
        out_type=jax.ShapeDtypeStruct((_B * _S, _D), jnp.float32),
        mesh=mesh,
        scratch_types=(
            [pltpu.VMEM((_B, _PPW), jnp.int32),
             pltpu.VMEM((_PPW, _D), jnp.float32)]
            + [pltpu.VMEM((_CHUNK, _D), jnp.float32)] * _NBUF
            + [pltpu.SemaphoreType.DMA] * (2 * _NBUF + 2)
        ),
    )
    out = run(x.astype(jnp.int32).reshape(-1), jnp.asarray(_PE),
              tok_emb_table)
    return out.reshape(_B, _S, _D)
